# baseline (speedup 1.0000x reference)
import functools

import jax
import jax.numpy as jnp
from jax import lax
from jax.experimental import pallas as pl
from jax.experimental.pallas import tpu as pltpu
from jax.experimental.pallas import tpu_sc as plsc

NC = 2
NS = 16
LANE = 16
IDXW = 128
NR = 3
BODY = NR * IDXW


def _chunks(total, step):
    out = []
    off = 0
    while off < total:
        c = min(step, total - off)
        out.append((off, c))
        off += c
    return out


def _mesh():
    return plsc.VectorSubcoreMesh(core_axis_name="c", subcore_axis_name="s",
                                  num_cores=NC, num_subcores=NS)


_CPARAMS = pltpu.CompilerParams(use_tc_tiling_on_sc=False)


def _plan(n_nodes, n_edges):
    half = n_nodes // 2
    trash = half
    zrows = (-(-(half + 1) // NS) + 7) // 8 * 8
    acc_rows = zrows * NS
    e_tile = n_edges // NS
    npair = e_tile // (2 * BODY)
    rem = e_tile - npair * 2 * BODY
    orows = -(-half // NS) // 8 * 8
    orows_last = half - orows * (NS - 1)
    assert rem % LANE == 0 and orows_last > 0 and orows_last % 8 == 0
    assert e_tile % 8 == 0 and n_edges % NS == 0
    return half, trash, zrows, acc_rows, e_tile, npair, rem, orows, orows_last


@functools.lru_cache(maxsize=None)
def _make_edge_kernel(n_nodes, n_edges, hid, table_rows):
    (half, trash, zrows, acc_rows, e_tile, npair, rem,
     orows, orows_last) = _plan(n_nodes, n_edges)
    del table_rows

    @functools.partial(
        pl.kernel,
        out_type=jax.ShapeDtypeStruct((n_nodes, hid), jnp.float32),
        mesh=_mesh(),
        scratch_types=[
            [pltpu.VMEM((BODY,), jnp.int32) for _ in range(2)],
            [pltpu.VMEM((BODY,), jnp.int32) for _ in range(2)],
            [pltpu.VMEM((BODY, hid), jnp.float32) for _ in range(2)],
            pltpu.VMEM_SHARED((acc_rows, hid), jnp.float32),
            pltpu.SemaphoreType.DMA,
            pltpu.SemaphoreType.DMA,
            pltpu.SemaphoreType.DMA,
        ],
        compiler_params=_CPARAMS,
    )
    def edge_kernel(t_hbm, s_hbm, r_hbm, z_hbm, out_hbm, sbuf, rbuf,
                    rows, acc_sh, isem, gsem, ssem):
        c = lax.axis_index("c")
        s = lax.axis_index("s")
        node_base = c * half

        pltpu.sync_copy(z_hbm, rows[0])
        zbase = s * zrows
        for off, cnt in _chunks(zrows, BODY):
            pltpu.sync_copy(rows[0].at[pl.ds(0, cnt)],
                            acc_sh.at[pl.ds(zbase + off, cnt)])
        plsc.subcore_barrier()

        ebase = s * e_tile

        def load_idx(e0, n_valid, p):
            nv8 = -(-n_valid // 8) * 8
            return (pltpu.async_copy(s_hbm.at[pl.ds(e0, nv8)],
                                     sbuf[p].at[pl.ds(0, nv8)], isem),
                    pltpu.async_copy(r_hbm.at[pl.ds(e0, nv8)],
                                     rbuf[p].at[pl.ds(0, nv8)], isem))

        def prep(n_valid, p):
            trash_v = jnp.full((LANE,), trash, jnp.int32)
            zero_v = jnp.zeros((LANE,), jnp.int32)
            for i in range(BODY // LANE):
                if i * LANE >= n_valid:
                    sbuf[p][pl.ds(i * LANE, LANE)] = zero_v
                    rbuf[p][pl.ds(i * LANE, LANE)] = trash_v
                else:
                    r = rbuf[p][pl.ds(i * LANE, LANE)]
                    rl = r - node_base
                    ok = (rl >= 0) & (rl < half)
                    rbuf[p][pl.ds(i * LANE, LANE)] = jnp.where(ok, rl, trash)

        def fire_gather(p):
            return pltpu.async_copy(t_hbm.at[sbuf[p]], rows[p], gsem)

        def fire_scatter(p):
            return pltpu.async_copy(rows[p], acc_sh.at[rbuf[p]], ssem,
                                    add=True)

        def pair(e0):
            iA = load_idx(e0, BODY, 0)
            iB = load_idx(e0 + BODY, BODY, 1)
            iA[0].wait()
            iA[1].wait()
            prep(BODY, 0)
            gA = fire_gather(0)
            iB[0].wait()
            iB[1].wait()
            prep(BODY, 1)
            gA.wait()
            sA = fire_scatter(0)
            gB = fire_gather(1)
            gB.wait()
            sA.wait()
            sB = fire_scatter(1)
            sB.wait()

        def body(b, carry):
            pair(ebase + b * 2 * BODY)
            return carry

        lax.fori_loop(0, npair, body, 0)
        for off, cnt in _chunks(rem, BODY):
            i0 = load_idx(ebase + npair * 2 * BODY + off, cnt, 0)
            i0[0].wait()
            i0[1].wait()
            prep(cnt, 0)
            fire_gather(0).wait()
            fire_scatter(0).wait()
        plsc.subcore_barrier()

        def writeout(n_out):
            obase = s * orows
            for off, cnt in _chunks(n_out, BODY):
                pltpu.sync_copy(acc_sh.at[pl.ds(obase + off, cnt)],
                                rows[0].at[pl.ds(0, cnt)])
                pltpu.sync_copy(rows[0].at[pl.ds(0, cnt)],
                                out_hbm.at[pl.ds(node_base + obase + off, cnt)])

        @pl.when(s < NS - 1)
        def _():
            writeout(orows)

        @pl.when(s == NS - 1)
        def _():
            writeout(orows_last)

    return edge_kernel


@functools.lru_cache(maxsize=None)
def _make_edge_kernel_bf16(n_nodes, n_edges, hid):
    trash = n_nodes
    zrows = (-(-(n_nodes + 1) // NS) + 7) // 8 * 8
    acc_rows = zrows * NS
    e_w = n_edges // (NC * NS)
    npair = e_w // (2 * BODY)
    rem = e_w - npair * 2 * BODY
    orows = -(-n_nodes // NS) // 8 * 8
    orows_last = n_nodes - orows * (NS - 1)
    assert rem % LANE == 0 and orows_last > 0 and orows_last % 8 == 0
    assert e_w % 8 == 0 and n_edges % (NC * NS) == 0

    @functools.partial(
        pl.kernel,
        out_type=jax.ShapeDtypeStruct((NC, n_nodes, hid), jnp.bfloat16),
        mesh=_mesh(),
        scratch_types=[
            [pltpu.VMEM((BODY,), jnp.int32) for _ in range(2)],
            [pltpu.VMEM((BODY,), jnp.int32) for _ in range(2)],
            [pltpu.VMEM((BODY, hid), jnp.bfloat16) for _ in range(2)],
            pltpu.VMEM_SHARED((acc_rows, hid), jnp.bfloat16),
            pltpu.SemaphoreType.DMA,
            pltpu.SemaphoreType.DMA,
            pltpu.SemaphoreType.DMA,
        ],
        compiler_params=_CPARAMS,
    )
    def edge_kernel(t_hbm, s_hbm, r_hbm, z_hbm, out_hbm, sbuf, rbuf,
                    rows, acc_sh, isem, gsem, ssem):
        c = lax.axis_index("c")
        s = lax.axis_index("s")

        pltpu.sync_copy(z_hbm, rows[0])
        zbase = s * zrows
        for off, cnt in _chunks(zrows, BODY):
            pltpu.sync_copy(rows[0].at[pl.ds(0, cnt)],
                            acc_sh.at[pl.ds(zbase + off, cnt)])
        plsc.subcore_barrier()

        ebase = (c * NS + s) * e_w

        def load_idx(e0, n_valid, p):
            nv8 = -(-n_valid // 8) * 8
            return (pltpu.async_copy(s_hbm.at[pl.ds(e0, nv8)],
                                     sbuf[p].at[pl.ds(0, nv8)], isem),
                    pltpu.async_copy(r_hbm.at[pl.ds(e0, nv8)],
                                     rbuf[p].at[pl.ds(0, nv8)], isem))

        def pad(n_valid, p):
            trash_v = jnp.full((LANE,), trash, jnp.int32)
            zero_v = jnp.zeros((LANE,), jnp.int32)
            for i in range(n_valid // LANE, BODY // LANE):
                sbuf[p][pl.ds(i * LANE, LANE)] = zero_v
                rbuf[p][pl.ds(i * LANE, LANE)] = trash_v

        def fire_gather(p):
            return pltpu.async_copy(t_hbm.at[sbuf[p]], rows[p], gsem)

        def fire_scatter(p):
            return pltpu.async_copy(rows[p], acc_sh.at[rbuf[p]], ssem,
                                    add=True)

        def pair(e0):
            iA = load_idx(e0, BODY, 0)
            iB = load_idx(e0 + BODY, BODY, 1)
            iA[0].wait()
            iA[1].wait()
            gA = fire_gather(0)
            iB[0].wait()
            iB[1].wait()
            gA.wait()
            sA = fire_scatter(0)
            gB = fire_gather(1)
            gB.wait()
            sA.wait()
            sB = fire_scatter(1)
            sB.wait()

        def body(b, carry):
            pair(ebase + b * 2 * BODY)
            return carry

        lax.fori_loop(0, npair, body, 0)
        for off, cnt in _chunks(rem, BODY):
            i0 = load_idx(ebase + npair * 2 * BODY + off, cnt, 0)
            i0[0].wait()
            i0[1].wait()
            pad(cnt, 0)
            fire_gather(0).wait()
            fire_scatter(0).wait()
        plsc.subcore_barrier()

        def writeout(n_out):
            obase = s * orows
            for off, cnt in _chunks(n_out, BODY):
                pltpu.sync_copy(acc_sh.at[pl.ds(obase + off, cnt)],
                                rows[0].at[pl.ds(0, cnt)])
                pltpu.sync_copy(rows[0].at[pl.ds(0, cnt)],
                                out_hbm.at[c].at[pl.ds(obase + off, cnt)])

        @pl.when(s < NS - 1)
        def _():
            writeout(orows)

        @pl.when(s == NS - 1)
        def _():
            writeout(orows_last)

    return edge_kernel


TCB = 2000


@functools.lru_cache(maxsize=None)
def _make_tc0(n, h):
    def body(pq, w0, b0r, wmb, wma_r, t0):
        h0 = jnp.dot(pq[...], w0[...], preferred_element_type=jnp.float32)
        h0 = h0 + b0r[...]
        t0[...] = (jnp.dot(h0, wmb[...], preferred_element_type=jnp.float32)
                   + wma_r[...]).astype(jnp.bfloat16)

    grid = (n // TCB,)
    return pl.pallas_call(
        body,
        grid=grid,
        in_specs=[
            pl.BlockSpec((TCB, 2), lambda i: (i, 0)),
            pl.BlockSpec((2, h), lambda i: (0, 0)),
            pl.BlockSpec((1, h), lambda i: (0, 0)),
            pl.BlockSpec((h, h), lambda i: (0, 0)),
            pl.BlockSpec((1, h), lambda i: (0, 0)),
        ],
        out_specs=pl.BlockSpec((TCB, h), lambda i: (i, 0)),
        out_shape=jax.ShapeDtypeStruct((n, h), jnp.bfloat16),
    )


@functools.lru_cache(maxsize=None)
def _make_tc1(n, h):
    def body(parts, pre, wm5, out):
        p = parts[...]
        acc = p[0].astype(jnp.float32) + p[1].astype(jnp.float32)
        acc = acc + jnp.dot(pre[...], wm5[...],
                            preferred_element_type=jnp.float32)
        out[...] = jnp.maximum(acc, 0.0)

    grid = (n // TCB,)
    return pl.pallas_call(
        body,
        grid=grid,
        in_specs=[
            pl.BlockSpec((2, TCB, h), lambda i: (0, i, 0)),
            pl.BlockSpec((TCB, h), lambda i: (i, 0)),
            pl.BlockSpec((h, h), lambda i: (0, 0)),
        ],
        out_specs=pl.BlockSpec((TCB, h), lambda i: (i, 0)),
        out_shape=jax.ShapeDtypeStruct((n, h), jnp.float32),
    )


@functools.lru_cache(maxsize=None)
def _make_tc2(n, h, with_next):
    def body(hin, v, grow, wna, wd, bdr, *rest):
        hh = jnp.dot(hin[...], wna[...], preferred_element_type=jnp.float32)
        hh = hh + grow[...]
        vn = v[...] + jnp.dot(hh, wd[...],
                              preferred_element_type=jnp.float32) + bdr[...]
        if with_next:
            wma, wmb, vout, tout = rest
            vout[...] = vn
            tout[...] = (jnp.dot(vn, wma[...],
                                 preferred_element_type=jnp.float32)
                         + jnp.dot(hh, wmb[...],
                                   preferred_element_type=jnp.float32)
                         ).astype(jnp.bfloat16)
        else:
            (vout,) = rest
            vout[...] = vn

    grid = (n // TCB,)
    in_specs = [
        pl.BlockSpec((TCB, h), lambda i: (i, 0)),
        pl.BlockSpec((TCB, 2), lambda i: (i, 0)),
        pl.BlockSpec((1, h), lambda i: (0, 0)),
        pl.BlockSpec((h, h), lambda i: (0, 0)),
        pl.BlockSpec((h, 2), lambda i: (0, 0)),
        pl.BlockSpec((1, 2), lambda i: (0, 0)),
    ]
    out_shape = [jax.ShapeDtypeStruct((n, 2), jnp.float32)]
    out_specs = [pl.BlockSpec((TCB, 2), lambda i: (i, 0))]
    if with_next:
        in_specs += [pl.BlockSpec((2, h), lambda i: (0, 0)),
                     pl.BlockSpec((h, h), lambda i: (0, 0))]
        out_shape.append(jax.ShapeDtypeStruct((n, h), jnp.bfloat16))
        out_specs.append(pl.BlockSpec((TCB, h), lambda i: (i, 0)))
    return pl.pallas_call(
        body,
        grid=grid,
        in_specs=in_specs,
        out_specs=out_specs,
        out_shape=out_shape,
    )


def kernel(P_Q_inj, senders, receivers, edge_features, params):
    N = P_Q_inj.shape[0]
    E = senders.shape[0]
    H = params["W0"].shape[1]
    D = edge_features.shape[1]

    s1 = senders.astype(jnp.int32)
    r1 = receivers.astype(jnp.int32)

    zeros_hb = jnp.zeros((BODY, H), jnp.bfloat16)

    edge_call = _make_edge_kernel_bf16(N, E, H)

    ef32 = jnp.concatenate(
        [edge_features.astype(jnp.bfloat16),
         jnp.ones((E, 1), jnp.bfloat16),
         jnp.zeros((E, H - D - 1), jnp.bfloat16)], axis=-1)
    eidx = jnp.arange(E, dtype=jnp.int32)
    pre_p = edge_call(ef32, eidx, r1, zeros_hb)
    pre = pre_p[0].astype(jnp.float32) + pre_p[1].astype(jnp.float32)
    efs, deg = pre[:, :D], pre[:, D:D + 1]

    del efs, deg

    layers = params["layers"]
    tc0 = _make_tc0(N, H)
    tc1 = _make_tc1(N, H)
    tc2n = _make_tc2(N, H, True)
    tc2l = _make_tc2(N, H, False)

    Wm0 = layers[0]["Wm"]
    T = tc0(P_Q_inj, params["W0"], params["b0"][None, :],
            Wm0[2:2 + H], Wm0[0][None, :])

    V = jnp.zeros_like(P_Q_inj).at[:, 0].set(1.0)
    g = jnp.zeros((1, H), jnp.float32)
    for i, lp in enumerate(layers):
        Wm5 = jnp.concatenate(
            [lp["Wm"][2 + H:], lp["bm"][None, :],
             jnp.zeros((H - D - 1, H), jnp.float32)], axis=0)
        parts = edge_call(T, s1, r1, zeros_hb)
        h = tc1(parts, pre, Wm5)
        nm = jnp.mean(h, axis=0, keepdims=True)
        g = jnp.concatenate([g, nm], axis=-1) @ lp["Wg"] + lp["bg"]
        grow = g @ lp["Wn"][H:] + lp["bn"][None, :]
        if i < len(layers) - 1:
            Wmn = layers[i + 1]["Wm"]
            V, T = tc2n(h, V, grow, lp["Wn"][:H], lp["Wd"],
                        lp["bd"][None, :], Wmn[:2], Wmn[2:2 + H])
        else:
            (V,) = tc2l(h, V, grow, lp["Wn"][:H], lp["Wd"],
                        lp["bd"][None, :])
    return V
